# baseline (speedup 1.0000x reference)
import numpy as np
import jax
import jax.numpy as jnp
from jax.experimental import pallas as pl
from jax.experimental.pallas import tpu as pltpu

_N = 2048
_F = _N // 2 + 1
_P = 1032
_C = 1024
_K = 64
_W = 512
_TW = 512


def _make_tables():
    t = np.arange(_N, dtype=np.float64)
    k = np.arange(_P, dtype=np.float64)
    tf = np.arange(_P, dtype=np.float64)
    ang = (2.0 * np.pi / _N) * np.outer(k, tf)
    cos_kt = np.cos(ang)
    sin_kt = np.sin(ang)
    valid = (np.arange(_P) < _F)[:, None].astype(np.float64)
    ct = cos_kt * valid
    st = -sin_kt * valid
    c = np.arange(_C, dtype=np.float64)
    ang2 = (2.0 * np.pi / _N) * np.outer(t, c)
    wc = np.where(c == 0, 1.0, 2.0) / _N
    ci = np.cos(ang2) * wc[None, :]
    si = -np.sin(ang2) * wc[None, :]
    f32 = np.float32
    return ct.astype(f32), st.astype(f32), ci.astype(f32), si.astype(f32)


_CT, _ST, _CI, _SI = _make_tables()


def _fwd_body(x_ref, xr_ref, ct_ref, st_ref, rm_ref, im_ref, magi_ref):
    xb = x_ref[0]
    rev = xr_ref[0]
    w = xb.shape[1]
    e_mid = xb[1:_N // 2] + rev[0:_N // 2 - 1]
    o_mid = xb[1:_N // 2] - rev[0:_N // 2 - 1]
    zero1 = jnp.zeros((1, w), jnp.float32)
    ee = jnp.concatenate(
        [xb[0:1], e_mid, rev[_N // 2 - 1:_N // 2],
         jnp.zeros((_P - _F, w), jnp.float32)], axis=0)
    oo = jnp.concatenate(
        [zero1, o_mid, jnp.zeros((_P - _F + 1, w), jnp.float32)], axis=0)
    rt = jnp.dot(ct_ref[...], ee,
                 preferred_element_type=jnp.float32,
                 precision=jax.lax.Precision.HIGHEST)
    it = jnp.dot(st_ref[...], oo,
                 preferred_element_type=jnp.float32,
                 precision=jax.lax.Precision.HIGHEST)
    mag = rt * rt + it * it
    magi_ref[...] = jax.lax.bitcast_convert_type(mag, jnp.int32)
    magi = magi_ref[...]

    def body(i, acc):
        cand = acc | (jnp.int32(1) << (jnp.int32(30) - i))
        pred = (magi >= cand).astype(jnp.float32)
        part = jnp.sum(pred.reshape(_P // 8, 8, _W), axis=0)
        cnt = jnp.sum(part, axis=0, keepdims=True)
        return jnp.where(cnt >= (_K - 0.5), cand, acc)

    thr = jax.lax.fori_loop(0, 31, body, jnp.zeros((1, _W), jnp.int32))
    mask = (magi >= thr).astype(jnp.float32)
    rm_ref[0] = rt * mask
    im_ref[0] = it * mask


def _inv_body(rm_ref, im_ref, ci_ref, si_ref, out_ref):
    nt = (((1,), (1,)), ((), ()))
    res = (
        jax.lax.dot_general(ci_ref[...], rm_ref[0].astype(jnp.bfloat16), nt,
                            preferred_element_type=jnp.float32)
        + jax.lax.dot_general(si_ref[...], im_ref[0].astype(jnp.bfloat16), nt,
                              preferred_element_type=jnp.float32)
    )
    out_ref[0] = res[:, :_F]


def kernel(x):
    batch, length, channels = x.shape
    nw = channels // _W
    grid = (batch, nw)
    ct = jnp.asarray(_CT)
    st = jnp.asarray(_ST)
    ci = jnp.asarray(_CI).astype(jnp.bfloat16)
    si = jnp.asarray(_SI).astype(jnp.bfloat16)

    xr = jnp.flip(x, axis=1)[:, : _N // 2, :]

    rm, im = pl.pallas_call(
        _fwd_body,
        grid=grid,
        in_specs=[
            pl.BlockSpec((1, _N // 2, _W), lambda b, c: (b, 0, c)),
            pl.BlockSpec((1, _N // 2, _W), lambda b, c: (b, 0, c)),
            pl.BlockSpec((_P, _P), lambda b, c: (0, 0)),
            pl.BlockSpec((_P, _P), lambda b, c: (0, 0)),
        ],
        out_specs=[
            pl.BlockSpec((1, _P, _W), lambda b, c: (b, 0, c)),
            pl.BlockSpec((1, _P, _W), lambda b, c: (b, 0, c)),
        ],
        out_shape=[
            jax.ShapeDtypeStruct((batch, _P, channels), jnp.float32),
            jax.ShapeDtypeStruct((batch, _P, channels), jnp.float32),
        ],
        scratch_shapes=[pltpu.VMEM((_P, _W), jnp.int32)],
    )(x, xr, ct, st)

    out = pl.pallas_call(
        _inv_body,
        grid=(batch, length // _TW),
        in_specs=[
            pl.BlockSpec((1, _P, _C), lambda b, t: (b, 0, 0)),
            pl.BlockSpec((1, _P, _C), lambda b, t: (b, 0, 0)),
            pl.BlockSpec((_TW, _C), lambda b, t: (t, 0)),
            pl.BlockSpec((_TW, _C), lambda b, t: (t, 0)),
        ],
        out_specs=pl.BlockSpec((1, _TW, _F), lambda b, t: (b, t, 0)),
        out_shape=jax.ShapeDtypeStruct((batch, length, _F), jnp.float32),
    )(rm, im, ci, si)
    return out


# speedup vs baseline: 1.3505x; 1.3505x over previous
import numpy as np
import jax
import jax.numpy as jnp
from jax.experimental import pallas as pl
from jax.experimental.pallas import tpu as pltpu

_N = 2048
_F = _N // 2 + 1
_P = 1032
_C = 1024
_K = 64
_W = 512
_TW = 512


def _make_tables():
    t = np.arange(_N, dtype=np.float64)
    k = np.arange(_P, dtype=np.float64)
    tf = np.arange(_P, dtype=np.float64)
    ang = (2.0 * np.pi / _N) * np.outer(k, tf)
    cos_kt = np.cos(ang)
    sin_kt = np.sin(ang)
    valid = (np.arange(_P) < _F)[:, None].astype(np.float64)
    ct = cos_kt * valid
    st = -sin_kt * valid
    c = np.arange(_C, dtype=np.float64)
    ang2 = (2.0 * np.pi / _N) * np.outer(t, c)
    wc = np.where(c == 0, 1.0, 2.0) / _N
    ci = np.cos(ang2) * wc[None, :]
    si = -np.sin(ang2) * wc[None, :]
    rp = np.zeros((_N // 2, _N // 2), dtype=np.float32)
    rp[np.arange(_N // 2), (_N // 2 - 1 - np.arange(_N // 2)) % (_N // 2)] = 1.0
    f32 = np.float32
    return (ct.astype(f32), st.astype(f32), ci.astype(f32), si.astype(f32),
            rp)


_CT, _ST, _CI, _SI, _RP = _make_tables()


def _fwd_body(x_ref, rp_ref, ct_ref, st_ref, rm_ref, im_ref, magi_ref):
    xb = x_ref[0]
    w = xb.shape[1]
    xb2 = xb[_N // 2:]
    h1 = xb2.astype(jnp.bfloat16)
    r1 = xb2 - h1.astype(jnp.float32)
    h2 = r1.astype(jnp.bfloat16)
    r2 = r1 - h2.astype(jnp.float32)
    h3 = r2.astype(jnp.bfloat16)
    rpb = rp_ref[...]
    rev = (jnp.dot(rpb, h1, preferred_element_type=jnp.float32)
           + jnp.dot(rpb, h2, preferred_element_type=jnp.float32)
           + jnp.dot(rpb, h3, preferred_element_type=jnp.float32))
    e_mid = xb[1:_N // 2] + rev[0:_N // 2 - 1]
    o_mid = xb[1:_N // 2] - rev[0:_N // 2 - 1]
    zero1 = jnp.zeros((1, w), jnp.float32)
    ee = jnp.concatenate(
        [xb[0:1], e_mid, rev[_N // 2 - 1:_N // 2],
         jnp.zeros((_P - _F, w), jnp.float32)], axis=0)
    oo = jnp.concatenate(
        [zero1, o_mid, jnp.zeros((_P - _F + 1, w), jnp.float32)], axis=0)
    rt = jnp.dot(ct_ref[...], ee,
                 preferred_element_type=jnp.float32,
                 precision=jax.lax.Precision.HIGHEST)
    it = jnp.dot(st_ref[...], oo,
                 preferred_element_type=jnp.float32,
                 precision=jax.lax.Precision.HIGHEST)
    mag = rt * rt + it * it
    magi_ref[...] = jax.lax.bitcast_convert_type(mag, jnp.int32)
    magi = magi_ref[...]

    def body(i, acc):
        cand = acc | (jnp.int32(1) << (jnp.int32(30) - i))
        pred = (magi >= cand).astype(jnp.float32)
        part = jnp.sum(pred.reshape(_P // 8, 8, _W), axis=0)
        cnt = jnp.sum(part, axis=0, keepdims=True)
        return jnp.where(cnt >= (_K - 0.5), cand, acc)

    thr = jax.lax.fori_loop(0, 31, body, jnp.zeros((1, _W), jnp.int32))
    mask = (magi >= thr).astype(jnp.float32)
    rm_ref[0] = rt * mask
    im_ref[0] = it * mask


def _inv_body(rm_ref, im_ref, ci_ref, si_ref, out_ref):
    nt = (((1,), (1,)), ((), ()))
    res = (
        jax.lax.dot_general(ci_ref[...], rm_ref[0].astype(jnp.bfloat16), nt,
                            preferred_element_type=jnp.float32)
        + jax.lax.dot_general(si_ref[...], im_ref[0].astype(jnp.bfloat16), nt,
                              preferred_element_type=jnp.float32)
    )
    out_ref[0] = res[:, :_F]


def kernel(x):
    batch, length, channels = x.shape
    nw = channels // _W
    grid = (batch, nw)
    ct = jnp.asarray(_CT)
    st = jnp.asarray(_ST)
    ci = jnp.asarray(_CI).astype(jnp.bfloat16)
    si = jnp.asarray(_SI).astype(jnp.bfloat16)

    rp = jnp.asarray(_RP).astype(jnp.bfloat16)

    rm, im = pl.pallas_call(
        _fwd_body,
        grid=grid,
        in_specs=[
            pl.BlockSpec((1, _N, _W), lambda b, c: (b, 0, c)),
            pl.BlockSpec((_N // 2, _N // 2), lambda b, c: (0, 0)),
            pl.BlockSpec((_P, _P), lambda b, c: (0, 0)),
            pl.BlockSpec((_P, _P), lambda b, c: (0, 0)),
        ],
        out_specs=[
            pl.BlockSpec((1, _P, _W), lambda b, c: (b, 0, c)),
            pl.BlockSpec((1, _P, _W), lambda b, c: (b, 0, c)),
        ],
        out_shape=[
            jax.ShapeDtypeStruct((batch, _P, channels), jnp.float32),
            jax.ShapeDtypeStruct((batch, _P, channels), jnp.float32),
        ],
        scratch_shapes=[pltpu.VMEM((_P, _W), jnp.int32)],
    )(x, rp, ct, st)

    out = pl.pallas_call(
        _inv_body,
        grid=(batch, length // _TW),
        in_specs=[
            pl.BlockSpec((1, _P, _C), lambda b, t: (b, 0, 0)),
            pl.BlockSpec((1, _P, _C), lambda b, t: (b, 0, 0)),
            pl.BlockSpec((_TW, _C), lambda b, t: (t, 0)),
            pl.BlockSpec((_TW, _C), lambda b, t: (t, 0)),
        ],
        out_specs=pl.BlockSpec((1, _TW, _F), lambda b, t: (b, t, 0)),
        out_shape=jax.ShapeDtypeStruct((batch, length, _F), jnp.float32),
    )(rm, im, ci, si)
    return out


# speedup vs baseline: 1.3779x; 1.0203x over previous
import numpy as np
import jax
import jax.numpy as jnp
from jax.experimental import pallas as pl
from jax.experimental.pallas import tpu as pltpu

_N = 2048
_F = _N // 2 + 1
_P = 1032
_C = 1024
_K = 64
_W = 512
_TW = 1024


def _make_tables():
    t = np.arange(_N, dtype=np.float64)
    k = np.arange(_P, dtype=np.float64)
    tf = np.arange(_P, dtype=np.float64)
    ang = (2.0 * np.pi / _N) * np.outer(k, tf)
    cos_kt = np.cos(ang)
    sin_kt = np.sin(ang)
    valid = (np.arange(_P) < _F)[:, None].astype(np.float64)
    ct = cos_kt * valid
    st = -sin_kt * valid
    c = np.arange(_C, dtype=np.float64)
    ang2 = (2.0 * np.pi / _N) * np.outer(t, c)
    wc = np.where(c == 0, 1.0, 2.0) / _N
    ci = np.cos(ang2) * wc[None, :]
    si = -np.sin(ang2) * wc[None, :]
    rp = np.zeros((_N // 2, _N // 2), dtype=np.float32)
    rp[np.arange(_N // 2), (_N // 2 - 1 - np.arange(_N // 2)) % (_N // 2)] = 1.0
    f32 = np.float32
    return (ct.astype(f32), st.astype(f32), ci.astype(f32), si.astype(f32),
            rp)


_CT, _ST, _CI, _SI, _RP = _make_tables()


def _fwd_body(x_ref, rp_ref, ct_ref, st_ref, rm_ref, im_ref, magi_ref):
    xb = x_ref[0]
    w = xb.shape[1]
    xb2 = xb[_N // 2:]
    h1 = xb2.astype(jnp.bfloat16)
    r1 = xb2 - h1.astype(jnp.float32)
    h2 = r1.astype(jnp.bfloat16)
    r2 = r1 - h2.astype(jnp.float32)
    h3 = r2.astype(jnp.bfloat16)
    rpb = rp_ref[...]
    rev = (jnp.dot(rpb, h1, preferred_element_type=jnp.float32)
           + jnp.dot(rpb, h2, preferred_element_type=jnp.float32)
           + jnp.dot(rpb, h3, preferred_element_type=jnp.float32))
    e_mid = xb[1:_N // 2] + rev[0:_N // 2 - 1]
    o_mid = xb[1:_N // 2] - rev[0:_N // 2 - 1]
    zero1 = jnp.zeros((1, w), jnp.float32)
    ee = jnp.concatenate(
        [xb[0:1], e_mid, rev[_N // 2 - 1:_N // 2],
         jnp.zeros((_P - _F, w), jnp.float32)], axis=0)
    oo = jnp.concatenate(
        [zero1, o_mid, jnp.zeros((_P - _F + 1, w), jnp.float32)], axis=0)
    rt = jnp.dot(ct_ref[...], ee,
                 preferred_element_type=jnp.float32,
                 precision=jax.lax.Precision.HIGHEST)
    it = jnp.dot(st_ref[...], oo,
                 preferred_element_type=jnp.float32,
                 precision=jax.lax.Precision.HIGHEST)
    mag = rt * rt + it * it
    magi_ref[...] = jax.lax.bitcast_convert_type(mag, jnp.int32)
    magi = magi_ref[...]

    def body(i, acc):
        cand = acc | (jnp.int32(1) << (jnp.int32(30) - i))
        pred = (magi >= cand).astype(jnp.float32)
        part = jnp.sum(pred.reshape(_P // 8, 8, _W), axis=0)
        cnt = jnp.sum(part, axis=0, keepdims=True)
        return jnp.where(cnt >= (_K - 0.5), cand, acc)

    thr = jax.lax.fori_loop(0, 31, body, jnp.zeros((1, _W), jnp.int32))
    mask = (magi >= thr).astype(jnp.float32)
    rm_ref[0] = rt * mask
    im_ref[0] = it * mask


def _inv_body(rm_ref, im_ref, ci_ref, si_ref, out_ref):
    nt = (((1,), (1,)), ((), ()))
    res = (
        jax.lax.dot_general(ci_ref[...], rm_ref[0].astype(jnp.bfloat16), nt,
                            preferred_element_type=jnp.float32)
        + jax.lax.dot_general(si_ref[...], im_ref[0].astype(jnp.bfloat16), nt,
                              preferred_element_type=jnp.float32)
    )
    out_ref[0] = res[:, :_F]


def kernel(x):
    batch, length, channels = x.shape
    nw = channels // _W
    grid = (batch, nw)
    ct = jnp.asarray(_CT)
    st = jnp.asarray(_ST)
    ci = jnp.asarray(_CI).astype(jnp.bfloat16)
    si = jnp.asarray(_SI).astype(jnp.bfloat16)

    rp = jnp.asarray(_RP).astype(jnp.bfloat16)

    rm, im = pl.pallas_call(
        _fwd_body,
        grid=grid,
        in_specs=[
            pl.BlockSpec((1, _N, _W), lambda b, c: (b, 0, c)),
            pl.BlockSpec((_N // 2, _N // 2), lambda b, c: (0, 0)),
            pl.BlockSpec((_P, _P), lambda b, c: (0, 0)),
            pl.BlockSpec((_P, _P), lambda b, c: (0, 0)),
        ],
        out_specs=[
            pl.BlockSpec((1, _P, _W), lambda b, c: (b, 0, c)),
            pl.BlockSpec((1, _P, _W), lambda b, c: (b, 0, c)),
        ],
        out_shape=[
            jax.ShapeDtypeStruct((batch, _P, channels), jnp.float32),
            jax.ShapeDtypeStruct((batch, _P, channels), jnp.float32),
        ],
        scratch_shapes=[pltpu.VMEM((_P, _W), jnp.int32)],
    )(x, rp, ct, st)

    out = pl.pallas_call(
        _inv_body,
        grid=(batch, length // _TW),
        in_specs=[
            pl.BlockSpec((1, _P, _C), lambda b, t: (b, 0, 0)),
            pl.BlockSpec((1, _P, _C), lambda b, t: (b, 0, 0)),
            pl.BlockSpec((_TW, _C), lambda b, t: (t, 0)),
            pl.BlockSpec((_TW, _C), lambda b, t: (t, 0)),
        ],
        out_specs=pl.BlockSpec((1, _TW, _F), lambda b, t: (b, t, 0)),
        out_shape=jax.ShapeDtypeStruct((batch, length, _F), jnp.float32),
    )(rm, im, ci, si)
    return out


# speedup vs baseline: 1.3792x; 1.0009x over previous
import numpy as np
import jax
import jax.numpy as jnp
from jax.experimental import pallas as pl
from jax.experimental.pallas import tpu as pltpu

_N = 2048
_F = _N // 2 + 1
_P = 1032
_C = 1024
_K = 64
_W = 512
_TW = 2048


def _make_tables():
    t = np.arange(_N, dtype=np.float64)
    k = np.arange(_P, dtype=np.float64)
    tf = np.arange(_P, dtype=np.float64)
    ang = (2.0 * np.pi / _N) * np.outer(k, tf)
    cos_kt = np.cos(ang)
    sin_kt = np.sin(ang)
    valid = (np.arange(_P) < _F)[:, None].astype(np.float64)
    ct = cos_kt * valid
    st = -sin_kt * valid
    c = np.arange(_C, dtype=np.float64)
    ang2 = (2.0 * np.pi / _N) * np.outer(t, c)
    wc = np.where(c == 0, 1.0, 2.0) / _N
    ci = np.cos(ang2) * wc[None, :]
    si = -np.sin(ang2) * wc[None, :]
    rp = np.zeros((_N // 2, _N // 2), dtype=np.float32)
    rp[np.arange(_N // 2), (_N // 2 - 1 - np.arange(_N // 2)) % (_N // 2)] = 1.0
    f32 = np.float32
    return (ct.astype(f32), st.astype(f32), ci.astype(f32), si.astype(f32),
            rp)


_CT, _ST, _CI, _SI, _RP = _make_tables()


def _fwd_body(x_ref, rp_ref, ct_ref, st_ref, rm_ref, im_ref, magi_ref):
    xb = x_ref[0]
    w = xb.shape[1]
    xb2 = xb[_N // 2:]
    h1 = xb2.astype(jnp.bfloat16)
    r1 = xb2 - h1.astype(jnp.float32)
    h2 = r1.astype(jnp.bfloat16)
    r2 = r1 - h2.astype(jnp.float32)
    h3 = r2.astype(jnp.bfloat16)
    rpb = rp_ref[...]
    rev = (jnp.dot(rpb, h1, preferred_element_type=jnp.float32)
           + jnp.dot(rpb, h2, preferred_element_type=jnp.float32)
           + jnp.dot(rpb, h3, preferred_element_type=jnp.float32))
    e_mid = xb[1:_N // 2] + rev[0:_N // 2 - 1]
    o_mid = xb[1:_N // 2] - rev[0:_N // 2 - 1]
    zero1 = jnp.zeros((1, w), jnp.float32)
    ee = jnp.concatenate(
        [xb[0:1], e_mid, rev[_N // 2 - 1:_N // 2],
         jnp.zeros((_P - _F, w), jnp.float32)], axis=0)
    oo = jnp.concatenate(
        [zero1, o_mid, jnp.zeros((_P - _F + 1, w), jnp.float32)], axis=0)
    rt = jnp.dot(ct_ref[...], ee,
                 preferred_element_type=jnp.float32,
                 precision=jax.lax.Precision.HIGHEST)
    it = jnp.dot(st_ref[...], oo,
                 preferred_element_type=jnp.float32,
                 precision=jax.lax.Precision.HIGHEST)
    mag = rt * rt + it * it
    magi_ref[...] = jax.lax.bitcast_convert_type(mag, jnp.int32)
    magi = magi_ref[...]

    def body(i, acc):
        cand = acc | (jnp.int32(1) << (jnp.int32(30) - i))
        pred = (magi >= cand).astype(jnp.float32)
        part = jnp.sum(pred.reshape(_P // 8, 8, _W), axis=0)
        cnt = jnp.sum(part, axis=0, keepdims=True)
        return jnp.where(cnt >= (_K - 0.5), cand, acc)

    thr = jax.lax.fori_loop(0, 31, body, jnp.zeros((1, _W), jnp.int32))
    mask = (magi >= thr).astype(jnp.float32)
    rm_ref[0] = rt * mask
    im_ref[0] = it * mask


def _inv_body(rm_ref, im_ref, ci_ref, si_ref, out_ref):
    nt = (((1,), (1,)), ((), ()))
    res = (
        jax.lax.dot_general(ci_ref[...], rm_ref[0].astype(jnp.bfloat16), nt,
                            preferred_element_type=jnp.float32)
        + jax.lax.dot_general(si_ref[...], im_ref[0].astype(jnp.bfloat16), nt,
                              preferred_element_type=jnp.float32)
    )
    out_ref[0] = res[:, :_F]


def kernel(x):
    batch, length, channels = x.shape
    nw = channels // _W
    grid = (batch, nw)
    ct = jnp.asarray(_CT)
    st = jnp.asarray(_ST)
    ci = jnp.asarray(_CI).astype(jnp.bfloat16)
    si = jnp.asarray(_SI).astype(jnp.bfloat16)

    rp = jnp.asarray(_RP).astype(jnp.bfloat16)

    rm, im = pl.pallas_call(
        _fwd_body,
        grid=grid,
        in_specs=[
            pl.BlockSpec((1, _N, _W), lambda b, c: (b, 0, c)),
            pl.BlockSpec((_N // 2, _N // 2), lambda b, c: (0, 0)),
            pl.BlockSpec((_P, _P), lambda b, c: (0, 0)),
            pl.BlockSpec((_P, _P), lambda b, c: (0, 0)),
        ],
        out_specs=[
            pl.BlockSpec((1, _P, _W), lambda b, c: (b, 0, c)),
            pl.BlockSpec((1, _P, _W), lambda b, c: (b, 0, c)),
        ],
        out_shape=[
            jax.ShapeDtypeStruct((batch, _P, channels), jnp.float32),
            jax.ShapeDtypeStruct((batch, _P, channels), jnp.float32),
        ],
        scratch_shapes=[pltpu.VMEM((_P, _W), jnp.int32)],
    )(x, rp, ct, st)

    out = pl.pallas_call(
        _inv_body,
        grid=(batch, length // _TW),
        in_specs=[
            pl.BlockSpec((1, _P, _C), lambda b, t: (b, 0, 0)),
            pl.BlockSpec((1, _P, _C), lambda b, t: (b, 0, 0)),
            pl.BlockSpec((_TW, _C), lambda b, t: (t, 0)),
            pl.BlockSpec((_TW, _C), lambda b, t: (t, 0)),
        ],
        out_specs=pl.BlockSpec((1, _TW, _F), lambda b, t: (b, t, 0)),
        out_shape=jax.ShapeDtypeStruct((batch, length, _F), jnp.float32),
    )(rm, im, ci, si)
    return out
